# baseline (speedup 1.0000x reference)
import functools

import jax
import jax.numpy as jnp
from jax.experimental import pallas as pl
from jax.experimental.pallas import tpu as pltpu


def _round_up(x, n):
    return ((x + n - 1) // n) * n


def _conv_t_kernel(x_ref, w_ref, ma_ref, mb_ref, s_ref, o_ref, *, wdim,
                   q_total, lt, ext):
    cin = x_ref.shape[1]
    xv = x_ref[0]
    sh = s_ref[...]
    head = wdim + 1
    for q0 in range(0, q_total, lt):
        s = q0 - head
        lo, hi = max(s, 0), min(s + ext, q_total)
        win = xv[:, lo:hi]
        if lo > s:
            win = jnp.concatenate(
                [jnp.zeros((cin, lo - s), jnp.bfloat16), win], axis=1)
        if hi < s + ext:
            win = jnp.concatenate(
                [win, jnp.zeros((cin, s + ext - hi), jnp.bfloat16)], axis=1)
        wa = win * ma_ref[...]
        wb = win * mb_ref[...]
        acc = jnp.zeros((o_ref.shape[1], lt), jnp.float32)
        for t in range(9):
            r, c = divmod(t, 3)
            d = r * wdim + c
            src = (wa, win, wb)[c]
            acc += jnp.dot(w_ref[t], src[:, d:d + lt],
                           preferred_element_type=jnp.float32)
        o_ref[0, :, q0:q0 + lt] = jnp.maximum(acc + sh, 0.0)


@jax.jit
def _conv_bn_relu(x, weight, gamma, beta, running_mean, running_var):
    n, cin, h, w = x.shape
    cout = weight.shape[0]
    eps = 1e-5
    q = h * w

    scale = gamma / jnp.sqrt(running_var + eps)
    shift = (beta - running_mean * scale).reshape(cout, 1)
    wt = (weight * scale[:, None, None, None]).astype(jnp.bfloat16)
    wt = jnp.transpose(wt, (2, 3, 0, 1)).reshape(9, cout, cin)

    lt = q
    for cand in (448, 512, 384, 256):
        if q % cand == 0:
            lt = cand
            break
    ext = _round_up(lt + 2 * w + 3, 128)

    lpos = jnp.arange(ext, dtype=jnp.int32) % w
    ma = (lpos != 0).astype(jnp.bfloat16).reshape(1, ext)
    mb = (lpos != 1).astype(jnp.bfloat16).reshape(1, ext)

    body = functools.partial(_conv_t_kernel, wdim=w, q_total=q, lt=lt,
                             ext=ext)
    out = pl.pallas_call(
        body,
        out_shape=jax.ShapeDtypeStruct((n, cout, q), jnp.float32),
        grid=(n,),
        in_specs=[
            pl.BlockSpec((1, cin, q), lambda i: (i, 0, 0)),
            pl.BlockSpec((9, cout, cin), lambda i: (0, 0, 0)),
            pl.BlockSpec((1, ext), lambda i: (0, 0)),
            pl.BlockSpec((1, ext), lambda i: (0, 0)),
            pl.BlockSpec((cout, 1), lambda i: (0, 0)),
        ],
        out_specs=pl.BlockSpec((1, cout, q), lambda i: (i, 0, 0)),
        compiler_params=pltpu.CompilerParams(
            dimension_semantics=("parallel",),
        ),
    )(x.reshape(n, cin, q).astype(jnp.bfloat16), wt, ma, mb, shift)

    return out.reshape(n, cout, h, w)


def kernel(x, weight, gamma, beta, running_mean, running_var):
    return _conv_bn_relu(x, weight, gamma, beta, running_mean, running_var)


# speedup vs baseline: 1.1136x; 1.1136x over previous
import functools

import jax
import jax.numpy as jnp
from jax.experimental import pallas as pl
from jax.experimental.pallas import tpu as pltpu


def _round_up(x, n):
    return ((x + n - 1) // n) * n


def _conv_t_kernel(x_ref, w_ref, ma_ref, mb_ref, s_ref, o_ref, *, wdim,
                   q_total, lt, ext):
    cin = x_ref.shape[1]
    sh = s_ref[...]
    head = wdim + 1
    for b in range(x_ref.shape[0]):
      xv = x_ref[b]
      for q0 in range(0, q_total, lt):
        s = q0 - head
        lo, hi = max(s, 0), min(s + ext, q_total)
        win = xv[:, lo:hi].astype(jnp.bfloat16)
        if lo > s:
            win = jnp.concatenate(
                [jnp.zeros((cin, lo - s), jnp.bfloat16), win], axis=1)
        if hi < s + ext:
            win = jnp.concatenate(
                [win, jnp.zeros((cin, s + ext - hi), jnp.bfloat16)], axis=1)
        wa = win * ma_ref[...]
        wb = win * mb_ref[...]
        acc = jnp.zeros((o_ref.shape[1], lt), jnp.float32)
        for t in range(9):
            r, c = divmod(t, 3)
            d = r * wdim + c
            src = (wa, win, wb)[c]
            acc += jnp.dot(w_ref[t], src[:, d:d + lt],
                           preferred_element_type=jnp.float32)
        o_ref[b, :, q0:q0 + lt] = jnp.maximum(acc + sh, 0.0)


@jax.jit
def _conv_bn_relu(x, weight, gamma, beta, running_mean, running_var):
    n, cin, h, w = x.shape
    cout = weight.shape[0]
    eps = 1e-5
    q = h * w

    scale = gamma / jnp.sqrt(running_var + eps)
    shift = (beta - running_mean * scale).reshape(cout, 1)
    wt = (weight * scale[:, None, None, None]).astype(jnp.bfloat16)
    wt = jnp.transpose(wt, (2, 3, 0, 1)).reshape(9, cout, cin)

    lt = q
    for cand in (448, 512, 384, 256):
        if q % cand == 0:
            lt = cand
            break
    ext = _round_up(lt + 2 * w + 3, 128)

    lpos = jnp.arange(ext, dtype=jnp.int32) % w
    ma = (lpos != 0).astype(jnp.bfloat16).reshape(1, ext)
    mb = (lpos != 1).astype(jnp.bfloat16).reshape(1, ext)

    nb = 2 if n % 2 == 0 else 1
    body = functools.partial(_conv_t_kernel, wdim=w, q_total=q, lt=lt,
                             ext=ext)
    out = pl.pallas_call(
        body,
        out_shape=jax.ShapeDtypeStruct((n, cout, q), jnp.float32),
        grid=(n // nb,),
        in_specs=[
            pl.BlockSpec((nb, cin, q), lambda i: (i, 0, 0)),
            pl.BlockSpec((9, cout, cin), lambda i: (0, 0, 0)),
            pl.BlockSpec((1, ext), lambda i: (0, 0)),
            pl.BlockSpec((1, ext), lambda i: (0, 0)),
            pl.BlockSpec((cout, 1), lambda i: (0, 0)),
        ],
        out_specs=pl.BlockSpec((nb, cout, q), lambda i: (i, 0, 0)),
        compiler_params=pltpu.CompilerParams(
            dimension_semantics=("parallel",),
        ),
    )(x.reshape(n, cin, q), wt, ma, mb, shift)

    return out.reshape(n, cout, h, w)


def kernel(x, weight, gamma, beta, running_mean, running_var):
    return _conv_bn_relu(x, weight, gamma, beta, running_mean, running_var)


# speedup vs baseline: 1.1212x; 1.0069x over previous
import functools

import jax
import jax.numpy as jnp
from jax.experimental import pallas as pl
from jax.experimental.pallas import tpu as pltpu


def _round_up(x, n):
    return ((x + n - 1) // n) * n


def _conv_t_kernel(x_ref, w_ref, ma_ref, mb_ref, s_ref, o_ref, *, wdim,
                   q_total, lt, ext):
    cin = x_ref.shape[1]
    sh = s_ref[...]
    head = wdim + 1
    for b in range(x_ref.shape[0]):
      xv = x_ref[b]
      for q0 in range(0, q_total, lt):
        s = q0 - head
        lo, hi = max(s, 0), min(s + ext, q_total)
        win = xv[:, lo:hi].astype(jnp.bfloat16)
        if lo > s:
            win = jnp.concatenate(
                [jnp.zeros((cin, lo - s), jnp.bfloat16), win], axis=1)
        if hi < s + ext:
            win = jnp.concatenate(
                [win, jnp.zeros((cin, s + ext - hi), jnp.bfloat16)], axis=1)
        wa = win * ma_ref[...]
        wb = win * mb_ref[...]
        acc = jnp.zeros((o_ref.shape[1], lt), jnp.float32)
        for t in range(9):
            r, c = divmod(t, 3)
            d = r * wdim + c
            src = (wa, win, wb)[c]
            acc += jnp.dot(w_ref[t], src[:, d:d + lt],
                           preferred_element_type=jnp.float32)
        o_ref[b, :, q0:q0 + lt] = jnp.maximum(acc + sh, 0.0)


@jax.jit
def _conv_bn_relu(x, weight, gamma, beta, running_mean, running_var):
    n, cin, h, w = x.shape
    cout = weight.shape[0]
    eps = 1e-5
    q = h * w

    scale = gamma / jnp.sqrt(running_var + eps)
    shift = (beta - running_mean * scale).reshape(cout, 1)
    wt = (weight * scale[:, None, None, None]).astype(jnp.bfloat16)
    wt = jnp.transpose(wt, (2, 3, 0, 1)).reshape(9, cout, cin)

    lt = q
    for cand in (448, 512, 384, 256):
        if q % cand == 0:
            lt = cand
            break
    ext = _round_up(lt + 2 * w + 3, 128)

    lpos = jnp.arange(ext, dtype=jnp.int32) % w
    ma = (lpos != 0).astype(jnp.bfloat16).reshape(1, ext)
    mb = (lpos != 1).astype(jnp.bfloat16).reshape(1, ext)

    nb = 4 if n % 4 == 0 else 1
    body = functools.partial(_conv_t_kernel, wdim=w, q_total=q, lt=lt,
                             ext=ext)
    out = pl.pallas_call(
        body,
        out_shape=jax.ShapeDtypeStruct((n, cout, q), jnp.float32),
        grid=(n // nb,),
        in_specs=[
            pl.BlockSpec((nb, cin, q), lambda i: (i, 0, 0)),
            pl.BlockSpec((9, cout, cin), lambda i: (0, 0, 0)),
            pl.BlockSpec((1, ext), lambda i: (0, 0)),
            pl.BlockSpec((1, ext), lambda i: (0, 0)),
            pl.BlockSpec((cout, 1), lambda i: (0, 0)),
        ],
        out_specs=pl.BlockSpec((nb, cout, q), lambda i: (i, 0, 0)),
        compiler_params=pltpu.CompilerParams(
            dimension_semantics=("parallel",),
        ),
    )(x.reshape(n, cin, q), wt, ma, mb, shift)

    return out.reshape(n, cout, h, w)


def kernel(x, weight, gamma, beta, running_mean, running_var):
    return _conv_bn_relu(x, weight, gamma, beta, running_mean, running_var)
